# Initial kernel scaffold; baseline (speedup 1.0000x reference)
import jax
import jax.numpy as jnp
from jax.experimental import pallas as pl


def kernel(edge_distance_vec, x_edge, edge_index, batch, W1, b1, W2, b2):
    raise NotImplementedError("write your pallas kernel here")


# speedup vs baseline: 1.1363x; 1.1363x over previous
import functools

import jax
import jax.numpy as jnp
import numpy as np
from jax.experimental import pallas as pl
from jax.experimental.pallas import tpu as pltpu

E = 1600000
N = 50000
G = 8
EMB = 16

_BLK = 4000


def _mlp_body(vec_ref, x_ref, w1_ref, b1_ref, w2_ref, b2_ref, out_ref):
    v = vec_ref[...]
    x = x_ref[...]
    a_parts = [v[:, a : a + 1] * x for a in range(3)]
    A = jnp.concatenate(a_parts, axis=1)
    eo_parts = [v[:, b : b + 1] * A for b in range(3)]
    EO = jnp.concatenate(eo_parts, axis=1)
    h = jnp.dot(EO, w1_ref[...], preferred_element_type=jnp.float32)
    h = h + b1_ref[...]
    h = h * jax.nn.sigmoid(h)
    out = jnp.dot(h, w2_ref[...], preferred_element_type=jnp.float32)
    out_ref[...] = out + b2_ref[...]


def _edge_mlp(vec, x, w1p, b1r, w2p, b2r):
    grid = (E // _BLK,)
    return pl.pallas_call(
        _mlp_body,
        grid=grid,
        in_specs=[
            pl.BlockSpec((_BLK, 3), lambda i: (i, 0)),
            pl.BlockSpec((_BLK, EMB), lambda i: (i, 0)),
            pl.BlockSpec((144, 144), lambda i: (0, 0)),
            pl.BlockSpec((1, 144), lambda i: (0, 0)),
            pl.BlockSpec((144, 16), lambda i: (0, 0)),
            pl.BlockSpec((1, 16), lambda i: (0, 0)),
        ],
        out_specs=pl.BlockSpec((_BLK, 16), lambda i: (i, 0)),
        out_shape=jax.ShapeDtypeStruct((E, 16), jnp.float32),
        compiler_params=pltpu.CompilerParams(
            dimension_semantics=("arbitrary",),
        ),
    )(vec, x, w1p, b1r, w2p, b2r)


_PERM = np.empty(144, dtype=np.int32)
for _b in range(3):
    for _a in range(3):
        for _i in range(16):
            _PERM[_b * 48 + _a * 16 + _i] = _i * 9 + _a * 3 + _b


def kernel(edge_distance_vec, x_edge, edge_index, batch, W1, b1, W2, b2):
    idx = edge_index.astype(jnp.int32)
    w1p = W1[jnp.asarray(_PERM), :]
    b1r = b1.reshape(1, 144)
    w2p = jnp.pad(W2, ((0, 0), (0, 16 - 9)))
    b2r = jnp.pad(b2, (0, 16 - 9)).at[9].set(1.0).reshape(1, 16)

    e9p = _edge_mlp(edge_distance_vec, x_edge, w1p, b1r, w2p, b2r)

    sums = jax.ops.segment_sum(e9p, idx, num_segments=N)
    cnt = jnp.maximum(sums[:, 9:10], 1.0)
    node_outer = sums[:, :9] / cnt

    bat = batch.astype(jnp.int32)
    gsums = jax.ops.segment_sum(node_outer, bat, num_segments=G)
    gcnt = jax.ops.segment_sum(jnp.ones((N, 1), jnp.float32), bat, num_segments=G)
    return gsums / jnp.maximum(gcnt, 1.0)


# speedup vs baseline: 2.5811x; 2.2715x over previous
import functools

import jax
import jax.numpy as jnp
import numpy as np
from jax import lax
from jax.experimental import pallas as pl
from jax.experimental.pallas import tpu as pltpu
from jax.experimental.pallas import tpu_sc as plsc

E = 1600000
N = 50000
G = 8
EMB = 16

_BLK = 4000


def _mlp_body(vec_ref, x_ref, w1_ref, b1_ref, w2_ref, b2_ref, out_ref):
    v = vec_ref[...]
    x = x_ref[...]
    a_parts = [v[:, a : a + 1] * x for a in range(3)]
    A = jnp.concatenate(a_parts, axis=1)
    eo_parts = [v[:, b : b + 1] * A for b in range(3)]
    EO = jnp.concatenate(eo_parts, axis=1)
    h = jnp.dot(EO, w1_ref[...], preferred_element_type=jnp.float32)
    h = h + b1_ref[...]
    h = h * jax.nn.sigmoid(h)
    out = jnp.dot(h, w2_ref[...], preferred_element_type=jnp.float32)
    out_ref[...] = out + b2_ref[...]


def _edge_mlp(vec, x, w1p, b1r, w2p, b2r):
    grid = (E // _BLK,)
    return pl.pallas_call(
        _mlp_body,
        grid=grid,
        in_specs=[
            pl.BlockSpec((_BLK, 3), lambda i: (i, 0)),
            pl.BlockSpec((_BLK, EMB), lambda i: (i, 0)),
            pl.BlockSpec((144, 144), lambda i: (0, 0)),
            pl.BlockSpec((1, 144), lambda i: (0, 0)),
            pl.BlockSpec((144, 16), lambda i: (0, 0)),
            pl.BlockSpec((1, 16), lambda i: (0, 0)),
        ],
        out_specs=pl.BlockSpec((_BLK, 16), lambda i: (i, 0)),
        out_shape=jax.ShapeDtypeStruct((E, 16), jnp.float32),
        compiler_params=pltpu.CompilerParams(
            dimension_semantics=("arbitrary",),
        ),
    )(vec, x, w1p, b1r, w2p, b2r)


_PERM = np.empty(144, dtype=np.int32)
for _b in range(3):
    for _a in range(3):
        for _i in range(16):
            _PERM[_b * 48 + _a * 16 + _i] = _i * 9 + _a * 3 + _b


_NB = E // 128
_NW = 32
_SB = 8
_NSB = _NB // _SB
_SB_PER_W = _NSB // _NW
_REM = _NSB - _SB_PER_W * _NW
_TAIL_ROW = _NSB * _SB
_NPS_A = 3128
_NPS_B = N - 15 * _NPS_A


def _sc_scatter_body(idx_hbm, e9_hbm, out_hbm, idx_v, rows_v, accum):
    c = lax.axis_index("c")
    s = lax.axis_index("s")
    wid = s * 2 + c

    def _zero(i, _):
        rows_v[i, :] = jnp.zeros((16,), jnp.float32)
        return 0
    lax.fori_loop(0, _SB * 128, _zero, 0)
    my_base = pl.multiple_of(s * _NPS_A, 8)
    for k in range(4):
        off = k * 1024
        size = [1024, 1024, 1024, 56][k]
        size_b = [1024, 1024, 1024, 8][k]
        @pl.when(s < 15)
        def _():
            pltpu.sync_copy(
                rows_v.at[pl.ds(0, size)],
                accum.at[pl.ds(my_base + off, size)],
            )
        @pl.when(s == 15)
        def _():
            pltpu.sync_copy(
                rows_v.at[pl.ds(0, size_b)],
                accum.at[pl.ds(my_base + off, size_b)],
            )
    plsc.subcore_barrier()

    base_sb = wid * _SB_PER_W + jnp.minimum(wid, _REM)

    def _do_sb(sb):
        row0 = pl.multiple_of(sb * _SB, _SB)
        erow0 = pl.multiple_of(sb * (_SB * 128), _SB * 128)
        pltpu.sync_copy(idx_hbm.at[pl.ds(row0, _SB)], idx_v)
        pltpu.sync_copy(e9_hbm.at[pl.ds(erow0, _SB * 128)], rows_v)
        for j in range(_SB):
            pltpu.sync_copy(
                rows_v.at[pl.ds(j * 128, 128)],
                accum.at[idx_v.at[j]],
                add=True,
            )

    def _chunk(t, _):
        _do_sb(base_sb + t)
        return 0

    lax.fori_loop(0, _SB_PER_W, _chunk, 0)

    @pl.when(wid < _REM)
    def _():
        _do_sb(base_sb + _SB_PER_W)

    @pl.when(wid == _NW - 1)
    def _():
        pltpu.sync_copy(idx_hbm.at[pl.ds(_TAIL_ROW, 4)], idx_v.at[pl.ds(0, 4)])
        pltpu.sync_copy(
            e9_hbm.at[pl.ds(_TAIL_ROW * 128, 512)], rows_v.at[pl.ds(0, 512)]
        )
        for j in range(4):
            pltpu.sync_copy(
                rows_v.at[pl.ds(j * 128, 128)],
                accum.at[idx_v.at[j]],
                add=True,
            )

    plsc.subcore_barrier()
    @pl.when(s < 15)
    def _():
        pltpu.sync_copy(
            accum.at[pl.ds(my_base, _NPS_A)],
            out_hbm.at[c].at[pl.ds(my_base, _NPS_A)],
        )
    @pl.when(s == 15)
    def _():
        pltpu.sync_copy(
            accum.at[pl.ds(15 * _NPS_A, _NPS_B)],
            out_hbm.at[c].at[pl.ds(15 * _NPS_A, _NPS_B)],
        )


def _sc_scatter(idx2d, e9):
    mesh = plsc.VectorSubcoreMesh(core_axis_name="c", subcore_axis_name="s")
    fn = functools.partial(
        pl.kernel,
        mesh=mesh,
        compiler_params=pltpu.CompilerParams(use_tc_tiling_on_sc=False),
        out_type=jax.ShapeDtypeStruct((2, N, 16), jnp.float32),
        scratch_types=[
            pltpu.VMEM((_SB, 128), jnp.int32),
            pltpu.VMEM((_SB * 128, 16), jnp.float32),
            pltpu.VMEM_SHARED((N, 16), jnp.float32),
        ],
    )(_sc_scatter_body)
    return fn(idx2d, e9)


_NBLK3 = 10
_B3 = N // _NBLK3


def _finish_body(part_ref, bat_ref, out_ref, gsum, gcnt):
    i = pl.program_id(0)

    @pl.when(i == 0)
    def _():
        gsum[...] = jnp.zeros((G, 16), jnp.float32)
        gcnt[...] = jnp.zeros((G, 16), jnp.float32)

    sums = part_ref[0] + part_ref[1]
    cnt = jnp.maximum(sums[:, 9:10], 1.0)
    node = sums / cnt
    bat = bat_ref[...]
    for g in range(G):
        m = (bat == g)
        gsum[g : g + 1, :] += jnp.sum(
            jnp.where(m, node, 0.0), axis=0, keepdims=True
        )
        gcnt[g : g + 1, :] += jnp.broadcast_to(
            jnp.sum(m.astype(jnp.float32)), (1, 16)
        )

    @pl.when(i == _NBLK3 - 1)
    def _():
        out_ref[...] = gsum[...] / jnp.maximum(gcnt[...], 1.0)


def _finish(partials, bat2d):
    return pl.pallas_call(
        _finish_body,
        grid=(_NBLK3,),
        in_specs=[
            pl.BlockSpec((2, _B3, 16), lambda i: (0, i, 0)),
            pl.BlockSpec((_B3, 1), lambda i: (i, 0)),
        ],
        out_specs=pl.BlockSpec((G, 16), lambda i: (0, 0)),
        out_shape=jax.ShapeDtypeStruct((G, 16), jnp.float32),
        scratch_shapes=[
            pltpu.VMEM((G, 16), jnp.float32),
            pltpu.VMEM((G, 16), jnp.float32),
        ],
        compiler_params=pltpu.CompilerParams(
            dimension_semantics=("arbitrary",),
        ),
    )(partials, bat2d)


def kernel(edge_distance_vec, x_edge, edge_index, batch, W1, b1, W2, b2):
    idx2d = edge_index.astype(jnp.int32).reshape(_NB, 128)
    w1p = W1[jnp.asarray(_PERM), :]
    b1r = b1.reshape(1, 144)
    w2p = jnp.pad(W2, ((0, 0), (0, 16 - 9)))
    b2r = jnp.pad(b2, (0, 16 - 9)).at[9].set(1.0).reshape(1, 16)

    e9p = _edge_mlp(edge_distance_vec, x_edge, w1p, b1r, w2p, b2r)
    partials = _sc_scatter(idx2d, e9p)
    bat2d = batch.astype(jnp.int32).reshape(N, 1)
    stress = _finish(partials, bat2d)
    return stress[:, :9]


# speedup vs baseline: 5.4434x; 2.1090x over previous
import functools

import jax
import jax.numpy as jnp
import numpy as np
from jax import lax
from jax.experimental import pallas as pl
from jax.experimental.pallas import tpu as pltpu
from jax.experimental.pallas import tpu_sc as plsc

E = 1600000
N = 50000
G = 8
EMB = 16

_BLK = 6400


def _mlp_body(vt_ref, xt_ref, w1_ref, b1_ref, w2_ref, b2_ref, out_ref):
    vt = vt_ref[...]
    xt = xt_ref[...]
    a_parts = [vt[a : a + 1, :] * xt for a in range(3)]
    AT = jnp.concatenate(a_parts, axis=0)
    eo_parts = [vt[b : b + 1, :] * AT for b in range(3)]
    EOT = jnp.concatenate(eo_parts, axis=0)
    h = jnp.dot(w1_ref[...], EOT, preferred_element_type=jnp.float32)
    h = h + b1_ref[...]
    h = h * jax.nn.sigmoid(h)
    out = jnp.dot(w2_ref[...], h, preferred_element_type=jnp.float32)
    out_ref[...] = out + b2_ref[...]


def _edge_mlp(vt, xt, w1t, b1c, w2t, b2c):
    grid = (E // _BLK,)
    return pl.pallas_call(
        _mlp_body,
        grid=grid,
        in_specs=[
            pl.BlockSpec((3, _BLK), lambda i: (0, i)),
            pl.BlockSpec((EMB, _BLK), lambda i: (0, i)),
            pl.BlockSpec((144, 144), lambda i: (0, 0)),
            pl.BlockSpec((144, 1), lambda i: (0, 0)),
            pl.BlockSpec((16, 144), lambda i: (0, 0)),
            pl.BlockSpec((16, 1), lambda i: (0, 0)),
        ],
        out_specs=pl.BlockSpec((16, _BLK), lambda i: (0, i)),
        out_shape=jax.ShapeDtypeStruct((16, E), jnp.float32),
        compiler_params=pltpu.CompilerParams(
            dimension_semantics=("arbitrary",),
        ),
    )(vt, xt, w1t, b1c, w2t, b2c)


_PERM = np.empty(144, dtype=np.int32)
for _b in range(3):
    for _a in range(3):
        for _i in range(16):
            _PERM[_b * 48 + _a * 16 + _i] = _i * 9 + _a * 3 + _b


_NB = E // 128
_NW = 32
_SB = 8
_NSB = _NB // _SB
_SB_PER_W = _NSB // _NW
_REM = _NSB - _SB_PER_W * _NW
_TAIL_ROW = _NSB * _SB
_NPS_A = 3128
_NPS_B = N - 15 * _NPS_A


def _sc_scatter_body(idx_hbm, e9_hbm, out_hbm, idx_v, rows_v, accum):
    c = lax.axis_index("c")
    s = lax.axis_index("s")
    wid = s * 2 + c

    def _zero(i, _):
        rows_v[i, :] = jnp.zeros((16,), jnp.float32)
        return 0
    lax.fori_loop(0, _SB * 128, _zero, 0)
    my_base = pl.multiple_of(s * _NPS_A, 8)
    for k in range(4):
        off = k * 1024
        size = [1024, 1024, 1024, 56][k]
        size_b = [1024, 1024, 1024, 8][k]
        @pl.when(s < 15)
        def _():
            pltpu.sync_copy(
                rows_v.at[pl.ds(0, size)],
                accum.at[pl.ds(my_base + off, size)],
            )
        @pl.when(s == 15)
        def _():
            pltpu.sync_copy(
                rows_v.at[pl.ds(0, size_b)],
                accum.at[pl.ds(my_base + off, size_b)],
            )
    plsc.subcore_barrier()

    base_sb = wid * _SB_PER_W + jnp.minimum(wid, _REM)

    def _do_sb(sb):
        row0 = pl.multiple_of(sb * _SB, _SB)
        erow0 = pl.multiple_of(sb * (_SB * 128), _SB * 128)
        pltpu.sync_copy(idx_hbm.at[pl.ds(row0, _SB)], idx_v)
        pltpu.sync_copy(e9_hbm.at[pl.ds(erow0, _SB * 128)], rows_v)
        for j in range(_SB):
            pltpu.sync_copy(
                rows_v.at[pl.ds(j * 128, 128)],
                accum.at[idx_v.at[j]],
                add=True,
            )

    def _chunk(t, _):
        _do_sb(base_sb + t)
        return 0

    lax.fori_loop(0, _SB_PER_W, _chunk, 0)

    @pl.when(wid < _REM)
    def _():
        _do_sb(base_sb + _SB_PER_W)

    @pl.when(wid == _NW - 1)
    def _():
        pltpu.sync_copy(idx_hbm.at[pl.ds(_TAIL_ROW, 4)], idx_v.at[pl.ds(0, 4)])
        pltpu.sync_copy(
            e9_hbm.at[pl.ds(_TAIL_ROW * 128, 512)], rows_v.at[pl.ds(0, 512)]
        )
        for j in range(4):
            pltpu.sync_copy(
                rows_v.at[pl.ds(j * 128, 128)],
                accum.at[idx_v.at[j]],
                add=True,
            )

    plsc.subcore_barrier()
    @pl.when(s < 15)
    def _():
        pltpu.sync_copy(
            accum.at[pl.ds(my_base, _NPS_A)],
            out_hbm.at[c].at[pl.ds(my_base, _NPS_A)],
        )
    @pl.when(s == 15)
    def _():
        pltpu.sync_copy(
            accum.at[pl.ds(15 * _NPS_A, _NPS_B)],
            out_hbm.at[c].at[pl.ds(15 * _NPS_A, _NPS_B)],
        )


def _sc_scatter(idx2d, e9):
    mesh = plsc.VectorSubcoreMesh(core_axis_name="c", subcore_axis_name="s")
    fn = functools.partial(
        pl.kernel,
        mesh=mesh,
        compiler_params=pltpu.CompilerParams(use_tc_tiling_on_sc=False),
        out_type=jax.ShapeDtypeStruct((2, N, 16), jnp.float32),
        scratch_types=[
            pltpu.VMEM((_SB, 128), jnp.int32),
            pltpu.VMEM((_SB * 128, 16), jnp.float32),
            pltpu.VMEM_SHARED((N, 16), jnp.float32),
        ],
    )(_sc_scatter_body)
    return fn(idx2d, e9)


_NBLK3 = 10
_B3 = N // _NBLK3


def _finish_body(part_ref, bat_ref, out_ref, gsum, gcnt):
    i = pl.program_id(0)

    @pl.when(i == 0)
    def _():
        gsum[...] = jnp.zeros((G, 16), jnp.float32)
        gcnt[...] = jnp.zeros((G, 16), jnp.float32)

    sums = part_ref[0] + part_ref[1]
    cnt = jnp.maximum(sums[:, 9:10], 1.0)
    node = sums / cnt
    bat = bat_ref[...]
    for g in range(G):
        m = (bat == g)
        gsum[g : g + 1, :] += jnp.sum(
            jnp.where(m, node, 0.0), axis=0, keepdims=True
        )
        gcnt[g : g + 1, :] += jnp.broadcast_to(
            jnp.sum(m.astype(jnp.float32)), (1, 16)
        )

    @pl.when(i == _NBLK3 - 1)
    def _():
        out_ref[...] = gsum[...] / jnp.maximum(gcnt[...], 1.0)


def _finish(partials, bat2d):
    return pl.pallas_call(
        _finish_body,
        grid=(_NBLK3,),
        in_specs=[
            pl.BlockSpec((2, _B3, 16), lambda i: (0, i, 0)),
            pl.BlockSpec((_B3, 1), lambda i: (i, 0)),
        ],
        out_specs=pl.BlockSpec((G, 16), lambda i: (0, 0)),
        out_shape=jax.ShapeDtypeStruct((G, 16), jnp.float32),
        scratch_shapes=[
            pltpu.VMEM((G, 16), jnp.float32),
            pltpu.VMEM((G, 16), jnp.float32),
        ],
        compiler_params=pltpu.CompilerParams(
            dimension_semantics=("arbitrary",),
        ),
    )(partials, bat2d)


def kernel(edge_distance_vec, x_edge, edge_index, batch, W1, b1, W2, b2):
    idx2d = edge_index.astype(jnp.int32).reshape(_NB, 128)
    w1t = W1[jnp.asarray(_PERM), :].T
    b1c = b1.reshape(144, 1)
    w2t = jnp.pad(W2, ((0, 0), (0, 16 - 9))).T
    b2c = jnp.pad(b2, (0, 16 - 9)).at[9].set(1.0).reshape(16, 1)

    e9t = _edge_mlp(edge_distance_vec.T, x_edge.T, w1t, b1c, w2t, b2c)
    partials = _sc_scatter(idx2d, e9t.T)
    bat2d = batch.astype(jnp.int32).reshape(N, 1)
    stress = _finish(partials, bat2d)
    return stress[:, :9]
